# baseline (speedup 1.0000x reference)
import functools

import jax
import jax.numpy as jnp
from jax import lax
from jax.experimental import pallas as pl
from jax.experimental.pallas import tpu as pltpu
from jax.experimental.pallas import tpu_sc as plsc

DIM = 64
BATCH = 16384
NEG = 5
VROWS = 500000

NC = 2
NS = 16
L = 16
NW = NC * NS
B_PER_W = BATCH // NW
CHUNK = 128
NCHUNK = B_PER_W // CHUNK


def _sc_scores(t_div, t_off, c_div, c_off, n_div, n_off, iemb2, oemb2):
    mesh = plsc.VectorSubcoreMesh(core_axis_name="c", subcore_axis_name="s")

    @functools.partial(
        pl.kernel,
        out_type=[
            jax.ShapeDtypeStruct((BATCH,), jnp.float32),
            jax.ShapeDtypeStruct((BATCH * NEG,), jnp.float32),
        ],
        mesh=mesh,
        compiler_params=pltpu.CompilerParams(
            needs_layout_passes=False, use_tc_tiling_on_sc=True),
        scratch_types=[
            pltpu.VMEM((CHUNK,), jnp.int32),
            pltpu.VMEM((CHUNK,), jnp.int32),
            pltpu.VMEM((CHUNK,), jnp.int32),
            pltpu.VMEM((CHUNK,), jnp.int32),
            pltpu.VMEM((NEG * CHUNK,), jnp.int32),
            pltpu.VMEM((NEG * CHUNK,), jnp.int32),
            pltpu.VMEM((CHUNK, 128), jnp.float32),
            pltpu.VMEM((CHUNK, 128), jnp.float32),
            pltpu.VMEM((NEG * CHUNK, 128), jnp.float32),
            pltpu.VMEM((CHUNK,), jnp.float32),
            pltpu.VMEM((NEG * CHUNK,), jnp.float32),
            pltpu.SemaphoreType.DMA,
        ],
    )
    def k(td_hbm, to_hbm, cd_hbm, co_hbm, nd_hbm, no_hbm, iemb_hbm, oemb_hbm,
          pos_hbm, negout_hbm,
          t_idx, t_po, c_idx, c_po, n_idx, n_po,
          t_rows, c_rows, n_rows, pos_v, neg_v, sem):
        wid = lax.axis_index("s") * NC + lax.axis_index("c")
        iota = lax.iota(jnp.int32, L)
        for ch in range(NCHUNK):
            base = wid * B_PER_W + ch * CHUNK
            stage = [
                pltpu.async_copy(td_hbm.at[pl.ds(base, CHUNK)], t_idx, sem),
                pltpu.async_copy(to_hbm.at[pl.ds(base, CHUNK)], t_po, sem),
                pltpu.async_copy(cd_hbm.at[pl.ds(base, CHUNK)], c_idx, sem),
                pltpu.async_copy(co_hbm.at[pl.ds(base, CHUNK)], c_po, sem),
                pltpu.async_copy(
                    nd_hbm.at[pl.ds(base * NEG, NEG * CHUNK)], n_idx, sem),
                pltpu.async_copy(
                    no_hbm.at[pl.ds(base * NEG, NEG * CHUNK)], n_po, sem),
            ]
            for cp in stage:
                cp.wait()
            cps = [
                pltpu.async_copy(iemb_hbm.at[t_idx], t_rows, sem),
                pltpu.async_copy(oemb_hbm.at[c_idx], c_rows, sem),
            ]
            for s in range(NEG):
                cps.append(pltpu.async_copy(
                    oemb_hbm.at[n_idx.at[pl.ds(s * CHUNK, CHUNK)]],
                    n_rows.at[pl.ds(s * CHUNK, CHUNK)], sem))
            for cp in cps:
                cp.wait()

            for blk in range(CHUNK // L):
                rows = blk * L + iota
                n_rowidx = [rows * NEG + kk for kk in range(NEG)]
                t_col0 = t_po[pl.ds(blk * L, L)]
                c_col0 = c_po[pl.ds(blk * L, L)]
                n_col0 = [plsc.load_gather(n_po, [n_rowidx[kk]])
                          for kk in range(NEG)]
                zero = jnp.zeros((L,), jnp.float32)

                def body(dd, carry, rows=rows, n_rowidx=n_rowidx,
                         t_col0=t_col0, c_col0=c_col0, n_col0=n_col0):
                    accp, accn = carry[0], list(carry[1:])
                    tv = plsc.load_gather(t_rows, [rows, t_col0 + dd])
                    cv = plsc.load_gather(c_rows, [rows, c_col0 + dd])
                    accp = accp + tv * cv
                    for kk in range(NEG):
                        nv = plsc.load_gather(
                            n_rows, [n_rowidx[kk], n_col0[kk] + dd])
                        accn[kk] = accn[kk] + tv * nv
                    return (accp, *accn)

                accs = lax.fori_loop(0, DIM, body, (zero,) * (1 + NEG))
                pos_v[pl.ds(blk * L, L)] = accs[0]
                for kk in range(NEG):
                    plsc.store_scatter(neg_v, [n_rowidx[kk]], accs[1 + kk])

            pltpu.sync_copy(pos_v, pos_hbm.at[pl.ds(base, CHUNK)])
            pltpu.sync_copy(neg_v, negout_hbm.at[pl.ds(base * NEG, NEG * CHUNK)])

    return k(t_div, t_off, c_div, c_off, n_div, n_off, iemb2, oemb2)


def _tc_loss(pos_scores, neg_scores):
    def body(p_ref, n_ref, pos_out, neg_out):
        p = p_ref[...]
        n = n_ref[...]

        def neg_logsig(x):
            return jnp.log(1.0 + jnp.exp(-jnp.abs(x))) - jnp.minimum(x, 0.0)

        pos_out[0, 0] = jnp.mean(neg_logsig(p))
        neg_out[0, 0] = jnp.mean(neg_logsig(-n))

    o1, o2 = pl.pallas_call(
        body,
        out_shape=[jax.ShapeDtypeStruct((1, 1), jnp.float32)] * 2,
        out_specs=[pl.BlockSpec(memory_space=pltpu.SMEM)] * 2,
    )(pos_scores.reshape(BATCH // 128, 128),
      neg_scores.reshape(BATCH * NEG // 128, 128))
    return o1[0, 0], o2[0, 0]


def kernel(target_words, context_words, negative_words, input_emb, output_emb):
    nf = negative_words.reshape(BATCH * NEG)
    t_div = jnp.right_shift(target_words, 1)
    c_div = jnp.right_shift(context_words, 1)
    n_div = jnp.right_shift(nf, 1)
    t_off = jnp.left_shift(jnp.bitwise_and(target_words, 1), 6)
    c_off = jnp.left_shift(jnp.bitwise_and(context_words, 1), 6)
    n_off = jnp.left_shift(jnp.bitwise_and(nf, 1), 6)
    iemb2 = input_emb.reshape(VROWS, 128)
    oemb2 = output_emb.reshape(VROWS, 128)
    pos_s, neg_s = _sc_scores(t_div, t_off, c_div, c_off, n_div, n_off,
                              iemb2, oemb2)
    return _tc_loss(pos_s, neg_s)


# speedup vs baseline: 1.1339x; 1.1339x over previous
import functools

import jax
import jax.numpy as jnp
from jax import lax
from jax.experimental import pallas as pl
from jax.experimental.pallas import tpu as pltpu
from jax.experimental.pallas import tpu_sc as plsc

VOCAB = 1000000
DIM = 64
BATCH = 16384
NEG = 5

NC = 2
NS = 16
L = 16
NW = NC * NS
B_PER_W = BATCH // NW
CHUNK = 128
NCHUNK = B_PER_W // CHUNK

RW = 2048


def _repack(embT):
    def body(in_ref, out_ref):
        out_ref[...] = jnp.concatenate(
            [in_ref[...].T, jnp.zeros((RW, DIM), jnp.float32)], axis=1)

    return pl.pallas_call(
        body,
        grid=(pl.cdiv(VOCAB, RW),),
        in_specs=[pl.BlockSpec((DIM, RW), lambda g: (0, g))],
        out_specs=pl.BlockSpec((RW, 128), lambda g: (g, 0)),
        out_shape=jax.ShapeDtypeStruct((VOCAB, 128), jnp.float32),
    )(embT)


def _sc_scores(target_words, context_words, neg_flat, ptab_i, ptab_o):
    mesh = plsc.VectorSubcoreMesh(core_axis_name="c", subcore_axis_name="s")

    @functools.partial(
        pl.kernel,
        out_type=[
            jax.ShapeDtypeStruct((BATCH,), jnp.float32),
            jax.ShapeDtypeStruct((BATCH * NEG,), jnp.float32),
        ],
        mesh=mesh,
        compiler_params=pltpu.CompilerParams(
            needs_layout_passes=False, use_tc_tiling_on_sc=True),
        scratch_types=[
            pltpu.VMEM((CHUNK,), jnp.int32),
            pltpu.VMEM((CHUNK,), jnp.int32),
            pltpu.VMEM((NEG * CHUNK,), jnp.int32),
            pltpu.VMEM((CHUNK, 128), jnp.float32),
            pltpu.VMEM((CHUNK, 128), jnp.float32),
            pltpu.VMEM((NEG * CHUNK, 128), jnp.float32),
            pltpu.VMEM((CHUNK,), jnp.float32),
            pltpu.VMEM((NEG * CHUNK,), jnp.float32),
            pltpu.SemaphoreType.DMA,
        ],
    )
    def k(tw_hbm, cw_hbm, nw_hbm, iemb_hbm, oemb_hbm, pos_hbm, negout_hbm,
          t_idx, c_idx, n_idx, t_rows, c_rows, n_rows, pos_v, neg_v, sem):
        wid = lax.axis_index("s") * NC + lax.axis_index("c")
        iota = lax.iota(jnp.int32, L)
        for ch in range(NCHUNK):
            base = wid * B_PER_W + ch * CHUNK
            stage = [
                pltpu.async_copy(tw_hbm.at[pl.ds(base, CHUNK)], t_idx, sem),
                pltpu.async_copy(cw_hbm.at[pl.ds(base, CHUNK)], c_idx, sem),
                pltpu.async_copy(
                    nw_hbm.at[pl.ds(base * NEG, NEG * CHUNK)], n_idx, sem),
            ]
            for cp in stage:
                cp.wait()
            cps = [
                pltpu.async_copy(iemb_hbm.at[t_idx], t_rows, sem),
                pltpu.async_copy(oemb_hbm.at[c_idx], c_rows, sem),
            ]
            for s in range(NEG):
                cps.append(pltpu.async_copy(
                    oemb_hbm.at[n_idx.at[pl.ds(s * CHUNK, CHUNK)]],
                    n_rows.at[pl.ds(s * CHUNK, CHUNK)], sem))
            for cp in cps:
                cp.wait()

            for blk in range(CHUNK // L):
                rows = blk * L + iota
                n_rowidx = [rows * NEG + kk for kk in range(NEG)]
                zero = jnp.zeros((L,), jnp.float32)

                def body(dd, carry, rows=rows, n_rowidx=n_rowidx):
                    accp, accn = carry[0], list(carry[1:])
                    col = jnp.full((L,), dd, jnp.int32)
                    tv = plsc.load_gather(t_rows, [rows, col])
                    cv = plsc.load_gather(c_rows, [rows, col])
                    accp = accp + tv * cv
                    for kk in range(NEG):
                        nv = plsc.load_gather(n_rows, [n_rowidx[kk], col])
                        accn[kk] = accn[kk] + tv * nv
                    return (accp, *accn)

                accs = lax.fori_loop(0, DIM, body, (zero,) * (1 + NEG))
                pos_v[pl.ds(blk * L, L)] = accs[0]
                for kk in range(NEG):
                    plsc.store_scatter(neg_v, [n_rowidx[kk]], accs[1 + kk])

            pltpu.sync_copy(pos_v, pos_hbm.at[pl.ds(base, CHUNK)])
            pltpu.sync_copy(neg_v, negout_hbm.at[pl.ds(base * NEG, NEG * CHUNK)])

    return k(target_words, context_words, neg_flat, ptab_i, ptab_o)


def _tc_loss(pos_scores, neg_scores):
    def body(p_ref, n_ref, pos_out, neg_out):
        p = p_ref[...]
        n = n_ref[...]

        def neg_logsig(x):
            return jnp.log(1.0 + jnp.exp(-jnp.abs(x))) - jnp.minimum(x, 0.0)

        pos_out[0, 0] = jnp.mean(neg_logsig(p))
        neg_out[0, 0] = jnp.mean(neg_logsig(-n))

    o1, o2 = pl.pallas_call(
        body,
        out_shape=[jax.ShapeDtypeStruct((1, 1), jnp.float32)] * 2,
        out_specs=[pl.BlockSpec(memory_space=pltpu.SMEM)] * 2,
    )(pos_scores.reshape(BATCH // 128, 128),
      neg_scores.reshape(BATCH * NEG // 128, 128))
    return o1[0, 0], o2[0, 0]


def kernel(target_words, context_words, negative_words, input_emb, output_emb):
    ptab_i = _repack(input_emb.T)
    ptab_o = _repack(output_emb.T)
    pos_s, neg_s = _sc_scores(target_words, context_words,
                              negative_words.reshape(BATCH * NEG),
                              ptab_i, ptab_o)
    return _tc_loss(pos_s, neg_s)
